# baseline (speedup 1.0000x reference)
import jax
import jax.numpy as jnp
from jax import lax
from jax.experimental import pallas as pl
from jax.experimental.pallas import tpu as pltpu


def _dsconv_kernel(x_ref, aux_ref, wp_ref, spw_ref, o_ref, *, H, WH, Co):
    x4 = x_ref[0]
    C = x4.shape[0]
    x = x4.reshape(C, WH)

    lane = lax.broadcasted_iota(jnp.int32, (1, WH), 1)
    hi = lax.rem(lane, H)
    mask_n = (hi != 0).astype(jnp.float32)
    mask_s = (hi != H - 1).astype(jnp.float32)

    aux = jnp.transpose(aux_ref[...])

    def col(k):
        return aux[:, k][:, None]

    s_n = pltpu.roll(x, 1, axis=1) * mask_n
    s_s = pltpu.roll(x, WH - 1, axis=1) * mask_s

    p_w = s_n * col(0) + x * col(3) + s_s * col(6)
    p_c = s_n * col(1) + x * col(4) + s_s * col(7)
    p_e = s_n * col(2) + x * col(5) + s_s * col(8)
    z = jnp.zeros((C, H), jnp.float32)
    q_w = jnp.concatenate([z, p_w[:, :WH - H]], axis=1)
    q_e = jnp.concatenate([p_e[:, H:], z], axis=1)
    dw = jnp.maximum(p_c + q_w + q_e + col(9), 0.0)

    pw = lax.dot_general(wp_ref[:, :Co].astype(jnp.bfloat16),
                         dw.astype(jnp.bfloat16),
                         (((0,), (0,)), ((), ())),
                         preferred_element_type=jnp.float32)
    spw = jnp.transpose(spw_ref[...])[:Co]
    o_ref[0] = jnp.maximum(pw + spw, 0.0)


def kernel(x, w_dw, s_dw, w_pw, s_pw):
    N, C, H, W = x.shape
    WH = W * H
    Co = w_pw.shape[1] // W

    aux = jnp.concatenate(
        [w_dw.reshape(9, W * C)[:, C:2 * C],
         s_dw[:, :C]], axis=0).astype(jnp.float32)
    spw2 = s_pw[:, :128]

    xt = jnp.transpose(x, (0, 1, 3, 2))

    flops = N * (18 * C * WH + 2 * C * Co * WH)
    bytes_accessed = 4 * N * WH * (C + Co) + aux.size * 4 + C * Co * 4

    out = pl.pallas_call(
        lambda *refs: _dsconv_kernel(*refs, H=H, WH=WH, Co=Co),
        out_shape=jax.ShapeDtypeStruct((N, Co, WH), jnp.float32),
        grid=(N,),
        in_specs=[
            pl.BlockSpec((1, C, W, H), lambda n: (n, 0, 0, 0)),
            pl.BlockSpec((10, C), lambda n: (0, 0)),
            pl.BlockSpec((C, 128), lambda n: (0, 0)),
            pl.BlockSpec((1, 128), lambda n: (0, 0)),
        ],
        out_specs=pl.BlockSpec((1, Co, WH), lambda n: (n, 0, 0)),
        compiler_params=pltpu.CompilerParams(
            dimension_semantics=("parallel",),
            vmem_limit_bytes=64 * 1024 * 1024),
        cost_estimate=pl.CostEstimate(flops=int(flops), transcendentals=0,
                                      bytes_accessed=int(bytes_accessed)),
    )(xt, aux, w_pw, spw2)

    return jnp.transpose(out.reshape(N, Co, W, H), (0, 1, 3, 2))


# speedup vs baseline: 2.6267x; 2.6267x over previous
import jax
import jax.numpy as jnp
from jax import lax
from jax.experimental import pallas as pl
from jax.experimental.pallas import tpu as pltpu


def _dsconv_kernel(x_ref, aux_ref, wp_ref, spw_ref, o_ref, *, H, WH, Co):
    x4 = x_ref[0]
    C = x4.shape[0]
    x = x4.reshape(C, WH)

    lane = lax.broadcasted_iota(jnp.int32, (1, WH), 1)
    hi = lax.rem(lane, H)
    mask_n = (hi != 0).astype(jnp.float32)
    mask_s = (hi != H - 1).astype(jnp.float32)

    aux = jnp.transpose(aux_ref[...])

    def col(k):
        return aux[:, k][:, None]

    s_n = pltpu.roll(x, 1, axis=1) * mask_n
    s_s = pltpu.roll(x, WH - 1, axis=1) * mask_s

    p_w = s_n * col(0) + x * col(3) + s_s * col(6)
    p_c = s_n * col(1) + x * col(4) + s_s * col(7)
    p_e = s_n * col(2) + x * col(5) + s_s * col(8)
    z = jnp.zeros((C, H), jnp.float32)
    q_w = jnp.concatenate([z, p_w[:, :WH - H]], axis=1)
    q_e = jnp.concatenate([p_e[:, H:], z], axis=1)
    dw = jnp.maximum(p_c + q_w + q_e + col(9), 0.0)

    pw = lax.dot_general(wp_ref[:, :Co].astype(jnp.bfloat16),
                         dw.astype(jnp.bfloat16),
                         (((0,), (0,)), ((), ())),
                         preferred_element_type=jnp.float32)
    spw = jnp.transpose(spw_ref[...])[:Co]
    pw = jnp.maximum(pw + spw, 0.0)
    o_ref[0] = pw.reshape(Co, WH // H, H)


def kernel(x, w_dw, s_dw, w_pw, s_pw):
    N, C, H, W = x.shape
    WH = W * H
    Co = w_pw.shape[1] // W

    aux = jnp.concatenate(
        [w_dw.reshape(9, W * C)[:, C:2 * C],
         s_dw[:, :C]], axis=0).astype(jnp.float32)
    spw2 = s_pw[:, :128]

    xt = jnp.transpose(x, (0, 1, 3, 2))

    flops = N * (18 * C * WH + 2 * C * Co * WH)
    bytes_accessed = 4 * N * WH * (C + Co) + aux.size * 4 + C * Co * 4

    out = pl.pallas_call(
        lambda *refs: _dsconv_kernel(*refs, H=H, WH=WH, Co=Co),
        out_shape=jax.ShapeDtypeStruct((N, Co, W, H), jnp.float32),
        grid=(N,),
        in_specs=[
            pl.BlockSpec((1, C, W, H), lambda n: (n, 0, 0, 0)),
            pl.BlockSpec((10, C), lambda n: (0, 0)),
            pl.BlockSpec((C, 128), lambda n: (0, 0)),
            pl.BlockSpec((1, 128), lambda n: (0, 0)),
        ],
        out_specs=pl.BlockSpec((1, Co, W, H), lambda n: (n, 0, 0, 0)),
        compiler_params=pltpu.CompilerParams(
            dimension_semantics=("parallel",),
            vmem_limit_bytes=64 * 1024 * 1024),
        cost_estimate=pl.CostEstimate(flops=int(flops), transcendentals=0,
                                      bytes_accessed=int(bytes_accessed)),
    )(xt, aux, w_pw, spw2)

    return jnp.transpose(out, (0, 1, 3, 2))


# speedup vs baseline: 2.7452x; 1.0451x over previous
import jax
import jax.numpy as jnp
from jax import lax
from jax.experimental import pallas as pl
from jax.experimental.pallas import tpu as pltpu


def _dsconv_kernel(x_ref, aux_ref, bm_ref, wp_ref, spw_ref, o_ref, *, H, WH, Co):
    x4 = x_ref[0]
    C = x4.shape[0]
    x = x4.reshape(C, WH)

    lane = lax.broadcasted_iota(jnp.int32, (1, WH), 1)
    hi = lax.rem(lane, H)
    mask_n = (hi != 0).astype(jnp.float32)
    mask_s = (hi != H - 1).astype(jnp.float32)

    aux = jnp.transpose(aux_ref[...])

    s_n = pltpu.roll(x, 1, axis=1) * mask_n
    s_s = pltpu.roll(x, WH - 1, axis=1) * mask_s

    stack = jnp.concatenate([s_n, x, s_s], axis=0).astype(jnp.bfloat16)
    p = lax.dot_general(bm_ref[...], stack,
                        (((0,), (0,)), ((), ())),
                        preferred_element_type=jnp.float32)

    z = jnp.zeros((C, H), jnp.float32)
    q_w = jnp.concatenate([z, p[:C, :WH - H]], axis=1)
    q_e = jnp.concatenate([p[2 * C:, H:], z], axis=1)
    dw = jnp.maximum(p[C:2 * C] + q_w + q_e + aux[:, 9][:, None], 0.0)

    pw = lax.dot_general(wp_ref[:, :Co].astype(jnp.bfloat16),
                         dw.astype(jnp.bfloat16),
                         (((0,), (0,)), ((), ())),
                         preferred_element_type=jnp.float32)
    spw = jnp.transpose(spw_ref[...])[:Co]
    pw = jnp.maximum(pw + spw, 0.0)
    o_ref[0] = pw.reshape(Co, WH // H, H)


def kernel(x, w_dw, s_dw, w_pw, s_pw):
    N, C, H, W = x.shape
    WH = W * H
    Co = w_pw.shape[1] // W

    aux = jnp.concatenate(
        [w_dw.reshape(9, W * C)[:, C:2 * C],
         s_dw[:, :C]], axis=0).astype(jnp.float32)
    spw2 = s_pw[:, :128]

    taps9 = w_dw.reshape(9, W * C)[:, C:2 * C]
    eye = jnp.eye(C, dtype=jnp.float32)
    bm = jnp.concatenate(
        [jnp.concatenate([eye * taps9[3 * j + i][None, :] for i in range(3)],
                         axis=1) for j in range(3)], axis=0).astype(jnp.bfloat16)

    xt = jnp.transpose(x, (0, 1, 3, 2))

    flops = N * (18 * C * WH + 2 * C * Co * WH)
    bytes_accessed = 4 * N * WH * (C + Co) + aux.size * 4 + C * Co * 4

    out = pl.pallas_call(
        lambda *refs: _dsconv_kernel(*refs, H=H, WH=WH, Co=Co),
        out_shape=jax.ShapeDtypeStruct((N, Co, W, H), jnp.float32),
        grid=(N,),
        in_specs=[
            pl.BlockSpec((1, C, W, H), lambda n: (n, 0, 0, 0)),
            pl.BlockSpec((10, C), lambda n: (0, 0)),
            pl.BlockSpec((3 * C, 3 * C), lambda n: (0, 0)),
            pl.BlockSpec((C, 128), lambda n: (0, 0)),
            pl.BlockSpec((1, 128), lambda n: (0, 0)),
        ],
        out_specs=pl.BlockSpec((1, Co, W, H), lambda n: (n, 0, 0, 0)),
        compiler_params=pltpu.CompilerParams(
            dimension_semantics=("parallel",),
            vmem_limit_bytes=64 * 1024 * 1024),
        cost_estimate=pl.CostEstimate(flops=int(flops), transcendentals=0,
                                      bytes_accessed=int(bytes_accessed)),
    )(xt, aux, bm, w_pw, spw2)

    return jnp.transpose(out, (0, 1, 3, 2))


# speedup vs baseline: 2.8872x; 1.0517x over previous
import jax
import jax.numpy as jnp
from jax import lax
from jax.experimental import pallas as pl
from jax.experimental.pallas import tpu as pltpu


def _dsconv_kernel(x_ref, aux_ref, bm_ref, wp_ref, spw_ref, o_ref, *, H, WH, Co):
    x4 = x_ref[0]
    C = x4.shape[0]
    x = x4.reshape(C, WH)

    lane = lax.broadcasted_iota(jnp.int32, (1, WH), 1)
    hi = lax.rem(lane, H)
    mask_n = (hi != 0).astype(jnp.float32)
    mask_s = (hi != H - 1).astype(jnp.float32)

    aux = jnp.transpose(aux_ref[...])

    xb = x.astype(jnp.bfloat16)
    s_n = pltpu.roll(xb, 1, axis=1) * mask_n.astype(jnp.bfloat16)
    s_s = pltpu.roll(xb, WH - 1, axis=1) * mask_s.astype(jnp.bfloat16)

    stack = jnp.concatenate([s_n, xb, s_s], axis=0)
    p = lax.dot_general(bm_ref[...], stack,
                        (((0,), (0,)), ((), ())),
                        preferred_element_type=jnp.float32)

    z = jnp.zeros((C, H), jnp.float32)
    q_w = jnp.concatenate([z, p[:C, :WH - H]], axis=1)
    q_e = jnp.concatenate([p[2 * C:, H:], z], axis=1)
    dw = jnp.maximum(p[C:2 * C] + q_w + q_e + aux[:, 9][:, None], 0.0)

    pw = lax.dot_general(wp_ref[:, :Co].astype(jnp.bfloat16),
                         dw.astype(jnp.bfloat16),
                         (((0,), (0,)), ((), ())),
                         preferred_element_type=jnp.float32)
    spw = jnp.transpose(spw_ref[...])[:Co]
    pw = jnp.maximum(pw + spw, 0.0)
    o_ref[0] = pw.reshape(Co, WH // H, H)


def kernel(x, w_dw, s_dw, w_pw, s_pw):
    N, C, H, W = x.shape
    WH = W * H
    Co = w_pw.shape[1] // W

    aux = jnp.concatenate(
        [w_dw.reshape(9, W * C)[:, C:2 * C],
         s_dw[:, :C]], axis=0).astype(jnp.float32)
    spw2 = s_pw[:, :128]

    taps9 = w_dw.reshape(9, W * C)[:, C:2 * C]
    eye = jnp.eye(C, dtype=jnp.float32)
    bm = jnp.concatenate(
        [jnp.concatenate([eye * taps9[3 * j + i][None, :] for i in range(3)],
                         axis=1) for j in range(3)], axis=0).astype(jnp.bfloat16)

    xt = jnp.transpose(x, (0, 1, 3, 2))

    flops = N * (18 * C * WH + 2 * C * Co * WH)
    bytes_accessed = 4 * N * WH * (C + Co) + aux.size * 4 + C * Co * 4

    out = pl.pallas_call(
        lambda *refs: _dsconv_kernel(*refs, H=H, WH=WH, Co=Co),
        out_shape=jax.ShapeDtypeStruct((N, Co, W, H), jnp.float32),
        grid=(N,),
        in_specs=[
            pl.BlockSpec((1, C, W, H), lambda n: (n, 0, 0, 0)),
            pl.BlockSpec((10, C), lambda n: (0, 0)),
            pl.BlockSpec((3 * C, 3 * C), lambda n: (0, 0)),
            pl.BlockSpec((C, 128), lambda n: (0, 0)),
            pl.BlockSpec((1, 128), lambda n: (0, 0)),
        ],
        out_specs=pl.BlockSpec((1, Co, W, H), lambda n: (n, 0, 0, 0)),
        compiler_params=pltpu.CompilerParams(
            dimension_semantics=("parallel",),
            vmem_limit_bytes=64 * 1024 * 1024),
        cost_estimate=pl.CostEstimate(flops=int(flops), transcendentals=0,
                                      bytes_accessed=int(bytes_accessed)),
    )(xt, aux, bm, w_pw, spw2)

    return jnp.transpose(out, (0, 1, 3, 2))
